# Initial kernel scaffold; baseline (speedup 1.0000x reference)
import jax
import jax.numpy as jnp
from jax.experimental import pallas as pl


def kernel(x, birth_table, gender_table, symp_tables, W1, b1, W2, b2):
    raise NotImplementedError("write your pallas kernel here")


# speedup vs baseline: 7.7813x; 7.7813x over previous
import functools

import jax
import jax.numpy as jnp
from jax import lax
from jax.experimental import pallas as pl
from jax.experimental.pallas import tpu as pltpu
from jax.experimental.pallas import tpu_sc as plsc

B = 16384
EMB = 64
HID = 128
NSYMP = 15
Q = 243
FROWS = 736
NC, NS, L = 2, 16, 16
NW = NC * NS
BPW = B // NW
GROUPS = BPW // L


def _fused_body(sall_ref, out_ref):
    r = lax.broadcasted_iota(jnp.int32, (FROWS, EMB), 0)
    k = lax.broadcasted_iota(jnp.int32, (FROWS, EMB), 1)
    rs = jnp.maximum(r - 2, 0)
    t = rs // Q
    c = rs % Q
    kk = jnp.maximum(k - 3, 0)
    il = kk // 3 - 5 * t
    d = kk % 3
    ilc = jnp.clip(il, 0, 4)
    p3 = jnp.where(ilc == 0, 81, jnp.where(ilc == 1, 27,
         jnp.where(ilc == 2, 9, jnp.where(ilc == 3, 3, 1))))
    digit = (c // p3) % 3
    symp_ok = (r >= 2) & (r < 731) & (k >= 3) & (il >= 0) & (il < 5) & (digit == d)
    gmask = (r < 2) & ((k == 0) | (k == 1 + r))
    M = jnp.where(gmask, 1.0 / 3.0, 0.0) + jnp.where(symp_ok, 1.0 / 45.0, 0.0)
    out_ref[...] = jnp.dot(M, sall_ref[...],
                           preferred_element_type=jnp.float32,
                           precision=lax.Precision.HIGHEST)


def _build_fused(sall):
    return pl.pallas_call(
        _fused_body,
        out_shape=jax.ShapeDtypeStruct((FROWS, EMB), jnp.float32),
    )(sall)


IDX_BLK = 512


def _idx_body(x_ref, out_ref):
    cv = x_ref[:, 1:17]
    m = lax.broadcasted_iota(jnp.int32, (16, 4), 0)
    tcol = lax.broadcasted_iota(jnp.int32, (16, 4), 1)
    mm = jnp.maximum(m - 1, 0)
    tt = mm // 5
    il = mm % 5
    p3 = jnp.where(il == 0, 81.0, jnp.where(il == 1, 27.0,
         jnp.where(il == 2, 9.0, jnp.where(il == 3, 3.0, 1.0))))
    sel = jnp.where((tcol == 0) & (m == 0), 1.0, 0.0)
    sel = sel + jnp.where((m >= 1) & (tcol == tt + 1), p3, 0.0)
    q = jnp.dot(cv, sel, preferred_element_type=jnp.float32,
                precision=lax.Precision.HIGHEST)
    qi = q.astype(jnp.int32)
    g = qi[:, 0:1]
    q0 = qi[:, 1:2]
    q1 = qi[:, 2:3]
    q2 = qi[:, 3:4]
    out_ref[...] = ((q2 * Q + q1) * Q + q0) * 2 + g


def _pack_idx(x):
    return pl.pallas_call(
        _idx_body,
        grid=(B // IDX_BLK,),
        in_specs=[pl.BlockSpec((IDX_BLK, 17), lambda i: (i, 0))],
        out_specs=pl.BlockSpec((IDX_BLK, 1), lambda i: (i, 0)),
        out_shape=jax.ShapeDtypeStruct((B, 1), jnp.int32),
    )(x)


def _sc_body(fused_hbm, idx_hbm, h_hbm, fused_v, idx_v, h_v):
    wid = lax.axis_index("s") * NC + lax.axis_index("c")
    base = wid * BPW
    pltpu.sync_copy(fused_hbm, fused_v)
    pltpu.sync_copy(idx_hbm.at[pl.ds(base, BPW)], idx_v)

    def group(gi, carry):
        off = gi * L
        p = idx_v[pl.ds(off, L)]
        g = jnp.bitwise_and(p, 1)
        qq = lax.shift_right_logical(p, 1)
        q0 = lax.rem(qq, jnp.int32(Q))
        q12 = lax.div(qq, jnp.int32(Q))
        q1 = lax.rem(q12, jnp.int32(Q))
        q2 = lax.div(q12, jnp.int32(Q))
        a0 = g * EMB
        a1 = (q0 + 2) * EMB
        a2 = (q1 + 2 + Q) * EMB
        a3 = (q2 + 2 + 2 * Q) * EMB
        for j in range(EMB):
            v = (plsc.load_gather(fused_v, [a0 + j])
                 + plsc.load_gather(fused_v, [a1 + j])
                 + plsc.load_gather(fused_v, [a2 + j])
                 + plsc.load_gather(fused_v, [a3 + j]))
            h_v[pl.ds(j * BPW + off, L)] = v
        return carry

    lax.fori_loop(0, GROUPS, group, 0)
    pltpu.sync_copy(h_v, h_hbm.at[pl.ds(base * EMB, BPW * EMB)])


@functools.lru_cache(maxsize=1)
def _sc_compute():
    mesh = plsc.VectorSubcoreMesh(core_axis_name="c", subcore_axis_name="s",
                                  num_cores=NC, num_subcores=NS)
    return pl.kernel(
        _sc_body,
        mesh=mesh,
        compiler_params=pltpu.CompilerParams(needs_layout_passes=False),
        out_type=jax.ShapeDtypeStruct((B * EMB,), jnp.float32),
        scratch_types=[
            pltpu.VMEM((FROWS * EMB,), jnp.float32),
            pltpu.VMEM((BPW,), jnp.int32),
            pltpu.VMEM((BPW * EMB,), jnp.float32),
        ],
    )


def _mlp_body(ht_ref, w1_ref, b1_ref, w2_ref, b2_ref, out_ref):
    ht = ht_ref[0]
    hh = lax.dot_general(w1_ref[...], ht, (((0,), (0,)), ((), ())),
                         preferred_element_type=jnp.float32)
    hh = jnp.maximum(hh + b1_ref[...][:, None], 0.0)
    o = lax.dot_general(w2_ref[...], hh, (((0,), (0,)), ((), ())),
                        preferred_element_type=jnp.float32)
    out_ref[...] = (o + b2_ref[...][:, None])[None]


def _mlp(ht, w1, b1, w2, b2):
    return pl.pallas_call(
        _mlp_body,
        grid=(NW,),
        in_specs=[
            pl.BlockSpec((1, EMB, BPW), lambda i: (i, 0, 0)),
            pl.BlockSpec((EMB, HID), lambda i: (0, 0)),
            pl.BlockSpec((HID,), lambda i: (0,)),
            pl.BlockSpec((HID, 1), lambda i: (0, 0)),
            pl.BlockSpec((1,), lambda i: (0,)),
        ],
        out_specs=pl.BlockSpec((1, 1, BPW), lambda i: (i, 0, 0)),
        out_shape=jax.ShapeDtypeStruct((NW, 1, BPW), jnp.float32),
    )(ht, w1, b1, w2, b2)


def kernel(x, birth_table, gender_table, symp_tables, W1, b1, W2, b2):
    sall = jnp.concatenate(
        [birth_table, gender_table, symp_tables.reshape(NSYMP * 3, EMB),
         jnp.zeros((EMB - 3 - NSYMP * 3, EMB), jnp.float32)], axis=0)
    fused = _build_fused(sall)
    idxp = _pack_idx(x)
    ht = _sc_compute()(fused.reshape(-1), idxp.reshape(-1))
    out = _mlp(ht.reshape(NW, EMB, BPW), W1, b1, W2, b2)
    return out.reshape(B, 1)


# speedup vs baseline: 12.5334x; 1.6107x over previous
import functools

import jax
import jax.numpy as jnp
from jax import lax
from jax.experimental import pallas as pl
from jax.experimental.pallas import tpu as pltpu
from jax.experimental.pallas import tpu_sc as plsc

B = 16384
EMB = 64
HID = 128
NSYMP = 15
Q = 243
NROWS = 1024
NC, NS, L = 2, 16, 16
NW = NC * NS
BPW = B // NW
GROUPS = BPW // L


def _fused_body(birth_ref, gender_ref, symp_ref, out_ref):
    ge = (birth_ref[...] + gender_ref[...]) * (1.0 / 3.0)
    s = symp_ref[...] * (1.0 / 45.0)

    def quint(t):
        T = s[15 * t: 15 * t + 3]
        n = 3
        for i in range(1, 5):
            a = s[15 * t + 3 * i: 15 * t + 3 * i + 3]
            T = (T[:, None, :] + a[None, :, :]).reshape(n * 3, EMB)
            n *= 3
        return T

    t0 = quint(0)
    t0g = (ge[:, None, :] + t0[None, :, :]).reshape(2 * Q, EMB)
    f = jnp.concatenate(
        [t0g, quint(1), quint(2), jnp.zeros((NROWS - 4 * Q, EMB), jnp.float32)],
        axis=0)
    out_ref[...] = f.T


def _build_fused(birth, gender, symp):
    return pl.pallas_call(
        _fused_body,
        out_shape=jax.ShapeDtypeStruct((EMB, NROWS), jnp.float32),
    )(birth, gender, symp)


def _sc_body(x_hbm, fused_hbm, h_hbm, x_v, fused_v, h_v):
    wid = lax.axis_index("s") * NC + lax.axis_index("c")
    base = wid * BPW
    pltpu.sync_copy(fused_hbm, fused_v)
    pltpu.sync_copy(x_hbm.at[pl.ds(base * 17, BPW * 17)], x_v)

    def group(gi, carry):
        off = gi * L
        rows = off + lax.iota(jnp.int32, L)

        rows17 = rows * 17

        def xg(col):
            return plsc.load_gather(x_v, [rows17 + col])

        def qv(t):
            acc = xg(2 + 5 * t)
            for i in range(1, 5):
                acc = acc * 3.0 + xg(2 + 5 * t + i)
            return acc

        r0 = (xg(1) * float(Q) + qv(0)).astype(jnp.int32)
        r1 = qv(1).astype(jnp.int32) + 2 * Q
        r2 = qv(2).astype(jnp.int32) + 3 * Q
        for j in range(EMB):
            fj = jnp.full((L,), j, jnp.int32)
            v = (plsc.load_gather(fused_v, [fj, r0])
                 + plsc.load_gather(fused_v, [fj, r1])
                 + plsc.load_gather(fused_v, [fj, r2]))
            h_v[pl.ds(j * BPW + off, L)] = v
        return carry

    lax.fori_loop(0, GROUPS, group, 0)
    pltpu.sync_copy(h_v, h_hbm.at[pl.ds(base * EMB, BPW * EMB)])


@functools.lru_cache(maxsize=1)
def _sc_compute():
    mesh = plsc.VectorSubcoreMesh(core_axis_name="c", subcore_axis_name="s",
                                  num_cores=NC, num_subcores=NS)
    return pl.kernel(
        _sc_body,
        mesh=mesh,
        compiler_params=pltpu.CompilerParams(needs_layout_passes=False),
        out_type=jax.ShapeDtypeStruct((B * EMB,), jnp.float32),
        scratch_types=[
            pltpu.VMEM((BPW * 17,), jnp.float32),
            pltpu.VMEM((EMB, NROWS), jnp.float32),
            pltpu.VMEM((BPW * EMB,), jnp.float32),
        ],
    )


MLP_CH = 4


def _mlp_body(ht_ref, w1_ref, b1_ref, w2_ref, b2_ref, out_ref):
    for w in range(MLP_CH):
        ht = ht_ref[w]
        hh = lax.dot_general(w1_ref[...], ht, (((0,), (0,)), ((), ())),
                             preferred_element_type=jnp.float32)
        hh = jnp.maximum(hh + b1_ref[...][:, None], 0.0)
        o = lax.dot_general(w2_ref[...], hh, (((0,), (0,)), ((), ())),
                            preferred_element_type=jnp.float32)
        out_ref[w] = o + b2_ref[...][:, None]


def _mlp(ht, w1, b1, w2, b2):
    return pl.pallas_call(
        _mlp_body,
        grid=(NW // MLP_CH,),
        in_specs=[
            pl.BlockSpec((MLP_CH, EMB, BPW), lambda i: (i, 0, 0)),
            pl.BlockSpec((EMB, HID), lambda i: (0, 0)),
            pl.BlockSpec((HID,), lambda i: (0,)),
            pl.BlockSpec((HID, 1), lambda i: (0, 0)),
            pl.BlockSpec((1,), lambda i: (0,)),
        ],
        out_specs=pl.BlockSpec((MLP_CH, 1, BPW), lambda i: (i, 0, 0)),
        out_shape=jax.ShapeDtypeStruct((NW, 1, BPW), jnp.float32),
    )(ht, w1, b1, w2, b2)


def kernel(x, birth_table, gender_table, symp_tables, W1, b1, W2, b2):
    fused = _build_fused(birth_table, gender_table,
                         symp_tables.reshape(NSYMP * 3, EMB))
    ht = _sc_compute()(x.reshape(-1), fused)
    out = _mlp(ht.reshape(NW, EMB, BPW), W1, b1, W2, b2)
    return out.reshape(B, 1)
